# baseline (speedup 1.0000x reference)
import functools

import jax
import jax.numpy as jnp
from jax import lax
from jax.experimental import pallas as pl
from jax.experimental.pallas import tpu as pltpu
from jax.experimental.pallas import tpu_sc as plsc

N_NODES = 10000
N_EDGES = 160000
D_IN = 500
D_HID = 32

NCORES = 2
NSUB = 16
NW = NCORES * NSUB
CHUNK = 128
CH = 40
EPAD = NW * CH * CHUNK
NPAD = 10112
RPT = NPAD // NSUB


MC = 4
NMEGA = CH // MC


def _make_edge_pass(d):
    mesh = plsc.VectorSubcoreMesh(core_axis_name="c", subcore_axis_name="s")
    MB = 512
    NMB = CH * CHUNK // MB

    @functools.partial(
        pl.kernel,
        out_type=jax.ShapeDtypeStruct((NCORES, NPAD, d), jnp.float32),
        mesh=mesh,
        scratch_types=[
            pltpu.VMEM((CH * CHUNK,), jnp.int32),
            pltpu.VMEM((CH * CHUNK,), jnp.int32),
            pltpu.VMEM((MB, d), jnp.float32),
            pltpu.VMEM((MB, d), jnp.float32),
            pltpu.VMEM_SHARED((NPAD, d), jnp.float32),
            pltpu.SemaphoreType.DMA,
            pltpu.SemaphoreType.DMA,
            pltpu.SemaphoreType.DMA,
            pltpu.SemaphoreType.DMA,
        ],
        compiler_params=pltpu.CompilerParams(use_tc_tiling_on_sc=False),
    )
    def k(g_hbm, src_hbm, dst_hbm, zeros_hbm, out_hbm, src_v, dst_v,
          buf_a, buf_b, acc, gs_a, gs_b, ss_a, ss_b):
        cid = lax.axis_index("c")
        sid = lax.axis_index("s")
        wid = cid * NSUB + sid
        pltpu.sync_copy(zeros_hbm.at[pl.ds(pl.multiple_of(sid * RPT, 8), RPT)],
                        acc.at[pl.ds(pl.multiple_of(sid * RPT, 8), RPT)])
        pltpu.sync_copy(src_hbm.at[pl.ds(wid * CH * CHUNK, CH * CHUNK)],
                        src_v)
        pltpu.sync_copy(dst_hbm.at[pl.ds(wid * CH * CHUNK, CH * CHUNK)],
                        dst_v)
        plsc.subcore_barrier()

        def fire_gather(m, buf, sem):
            pltpu.async_copy(g_hbm.at[src_v.at[pl.ds(m * MB, MB)]], buf, sem)

        def wait_gather(m, buf, sem):
            pltpu.make_async_copy(g_hbm.at[src_v.at[pl.ds(m * MB, MB)]], buf,
                                  sem).wait()

        def fire_scatter(m, buf, sem):
            pltpu.async_copy(buf, acc.at[dst_v.at[pl.ds(m * MB, MB)]], sem,
                             add=True)

        def wait_scatter(m, buf, sem):
            pltpu.make_async_copy(buf, acc.at[dst_v.at[pl.ds(m * MB, MB)]],
                                  sem).wait()

        fire_gather(0, buf_a, gs_a)
        fire_gather(1, buf_b, gs_b)

        def body(i, carry):
            m_a = i * 2
            m_b = i * 2 + 1
            wait_gather(m_a, buf_a, gs_a)
            fire_scatter(m_a, buf_a, ss_a)
            wait_gather(m_b, buf_b, gs_b)
            fire_scatter(m_b, buf_b, ss_b)
            wait_scatter(m_a, buf_a, ss_a)

            @pl.when(i < NMB // 2 - 1)
            def _():
                fire_gather(m_a + 2, buf_a, gs_a)

            wait_scatter(m_b, buf_b, ss_b)

            @pl.when(i < NMB // 2 - 1)
            def _():
                fire_gather(m_b + 2, buf_b, gs_b)

            return carry

        lax.fori_loop(0, NMB // 2, body, 0)
        plsc.subcore_barrier()
        pltpu.sync_copy(acc.at[pl.ds(pl.multiple_of(sid * RPT, 8), RPT)],
                        out_hbm.at[cid].at[pl.ds(pl.multiple_of(sid * RPT, 8), RPT)])

    return k


def _make_deg_pass(d):
    mesh = plsc.VectorSubcoreMesh(core_axis_name="c", subcore_axis_name="s")

    @functools.partial(
        pl.kernel,
        out_type=jax.ShapeDtypeStruct((NCORES, NPAD, d), jnp.float32),
        mesh=mesh,
        scratch_types=[
            pltpu.VMEM((CH, CHUNK), jnp.int32),
            pltpu.VMEM((MC, CHUNK, d), jnp.float32),
            pltpu.VMEM_SHARED((NPAD, d), jnp.float32),
            pltpu.SemaphoreType.DMA,
        ],
        compiler_params=pltpu.CompilerParams(use_tc_tiling_on_sc=False),
    )
    def k(ones_hbm, dst_hbm, zeros_hbm, out_hbm, dst_v, ones_v, acc, sem):
        cid = lax.axis_index("c")
        sid = lax.axis_index("s")
        wid = cid * NSUB + sid
        pltpu.sync_copy(zeros_hbm.at[pl.ds(pl.multiple_of(sid * RPT, 8), RPT)],
                        acc.at[pl.ds(pl.multiple_of(sid * RPT, 8), RPT)])
        pltpu.sync_copy(dst_hbm.at[pl.ds(wid * CH, CH)], dst_v)
        pltpu.sync_copy(ones_hbm, ones_v)
        plsc.subcore_barrier()

        def fire(c, carry):
            pltpu.async_copy(ones_v.at[0], acc.at[dst_v.at[c]], sem, add=True)
            return carry

        def drain(c, carry):
            pltpu.make_async_copy(ones_v.at[0], acc.at[dst_v.at[c]],
                                  sem).wait()
            return carry

        lax.fori_loop(0, CH, fire, 0)
        lax.fori_loop(0, CH, drain, 0)
        plsc.subcore_barrier()
        pltpu.sync_copy(acc.at[pl.ds(pl.multiple_of(sid * RPT, 8), RPT)],
                        out_hbm.at[cid].at[pl.ds(pl.multiple_of(sid * RPT, 8), RPT)])

    return k


_edge_pass_32 = _make_edge_pass(D_HID)
_deg_pass = _make_deg_pass(8)


def _mm1_body(x_ref, w_ref, h_ref):
    h_ref[...] = jnp.dot(x_ref[...], w_ref[...],
                         preferred_element_type=jnp.float32)


def _mm1(x, w1):
    bm = 1000
    return pl.pallas_call(
        _mm1_body,
        grid=(N_NODES // bm,),
        in_specs=[
            pl.BlockSpec((bm, D_IN), lambda i: (i, 0)),
            pl.BlockSpec((D_IN, D_HID), lambda i: (0, 0)),
        ],
        out_specs=pl.BlockSpec((bm, D_HID), lambda i: (i, 0)),
        out_shape=jax.ShapeDtypeStruct((N_NODES, D_HID), jnp.float32),
    )(x, w1)


def _scale_body(h_ref, degp_ref, g_ref, dinv_ref):
    dp = degp_ref[...]
    deg = dp[0, :N_NODES, 0:1] + dp[1, :N_NODES, 0:1] + 1.0
    dinv = 1.0 / jnp.sqrt(deg)
    dinv_ref[...] = dinv
    g_ref[...] = h_ref[...] * dinv


def _scale(h1, degp):
    return pl.pallas_call(
        _scale_body,
        in_specs=[
            pl.BlockSpec((N_NODES, D_HID), lambda: (0, 0)),
            pl.BlockSpec((NCORES, NPAD, 8), lambda: (0, 0, 0)),
        ],
        out_specs=[
            pl.BlockSpec((N_NODES, D_HID), lambda: (0, 0)),
            pl.BlockSpec((N_NODES, 1), lambda: (0, 0)),
        ],
        out_shape=[
            jax.ShapeDtypeStruct((N_NODES, D_HID), jnp.float32),
            jax.ShapeDtypeStruct((N_NODES, 1), jnp.float32),
        ],
    )(h1, degp)


def _mid_body(part_ref, g1_ref, dinv_ref, b1_ref, f_ref):
    p = part_ref[...]
    dinv = dinv_ref[...]
    s1 = g1_ref[...] + p[0, :N_NODES, :] + p[1, :N_NODES, :]
    e = jnp.tanh(s1 * dinv + b1_ref[...])
    f_ref[...] = e * dinv


def _mid(part1, g1, dinv, b1):
    return pl.pallas_call(
        _mid_body,
        in_specs=[
            pl.BlockSpec((NCORES, NPAD, D_HID), lambda: (0, 0, 0)),
            pl.BlockSpec((N_NODES, D_HID), lambda: (0, 0)),
            pl.BlockSpec((N_NODES, 1), lambda: (0, 0)),
            pl.BlockSpec((1, D_HID), lambda: (0, 0)),
        ],
        out_specs=pl.BlockSpec((N_NODES, D_HID), lambda: (0, 0)),
        out_shape=jax.ShapeDtypeStruct((N_NODES, D_HID), jnp.float32),
    )(part1, g1, dinv, b1)


def _fin_body(part_ref, f_ref, dinv_ref, wcat_ref, bcat_ref, mu_ref, ls_ref):
    p = part_ref[...]
    s2 = f_ref[...] + p[0, :N_NODES, :] + p[1, :N_NODES, :]
    h2 = jnp.dot(s2, wcat_ref[...], preferred_element_type=jnp.float32)
    o = h2 * dinv_ref[...] + bcat_ref[...]
    mu_ref[...] = o[:, :D_HID]
    ls_ref[...] = o[:, D_HID:]


def _fin(part2, f, dinv, wcat, bcat):
    return pl.pallas_call(
        _fin_body,
        in_specs=[
            pl.BlockSpec((NCORES, NPAD, D_HID), lambda: (0, 0, 0)),
            pl.BlockSpec((N_NODES, D_HID), lambda: (0, 0)),
            pl.BlockSpec((N_NODES, 1), lambda: (0, 0)),
            pl.BlockSpec((D_HID, 2 * D_HID), lambda: (0, 0)),
            pl.BlockSpec((1, 2 * D_HID), lambda: (0, 0)),
        ],
        out_specs=[
            pl.BlockSpec((N_NODES, D_HID), lambda: (0, 0)),
            pl.BlockSpec((N_NODES, D_HID), lambda: (0, 0)),
        ],
        out_shape=[
            jax.ShapeDtypeStruct((N_NODES, D_HID), jnp.float32),
            jax.ShapeDtypeStruct((N_NODES, D_HID), jnp.float32),
        ],
    )(part2, f, dinv, wcat, bcat)


def kernel(x, edge_index, W1, b1, Wmu, bmu, Wls, bls):
    ei = edge_index.astype(jnp.int32)
    npad = EPAD - N_EDGES
    pad_dst = N_NODES + jnp.arange(npad, dtype=jnp.int32) % (NPAD - N_NODES)
    src1 = jnp.concatenate([ei[0], jnp.zeros((npad,), jnp.int32)])
    dst1 = jnp.concatenate([ei[1], pad_dst])
    dst2d = dst1.reshape(NW * CH, CHUNK)

    z8 = jnp.zeros((NPAD, 8), jnp.float32)
    z32 = jnp.zeros((NPAD, D_HID), jnp.float32)
    ones8 = jnp.ones((MC, CHUNK, 8), jnp.float32)
    wcat = jnp.concatenate([Wmu, Wls], axis=1)
    bcat = jnp.concatenate([bmu, bls]).reshape(1, 2 * D_HID)

    h1 = _mm1(x, W1)
    degp = _deg_pass(ones8, dst2d, z8)
    g1, dinv = _scale(h1, degp)
    part1 = _edge_pass_32(g1, src1, dst1, z32)
    f = _mid(part1, g1, dinv, b1.reshape(1, D_HID))
    part2 = _edge_pass_32(f, src1, dst1, z32)
    mu, ls = _fin(part2, f, dinv, wcat, bcat)
    return mu, ls


# speedup vs baseline: 1.2400x; 1.2400x over previous
import functools

import jax
import jax.numpy as jnp
from jax import lax
from jax.experimental import pallas as pl
from jax.experimental.pallas import tpu as pltpu
from jax.experimental.pallas import tpu_sc as plsc

N_NODES = 10000
N_EDGES = 160000
D_IN = 500
D_HID = 32

NCORES = 2
NSUB = 16
NW = NCORES * NSUB
CHUNK = 128
CH = 40
EPAD = NW * CH * CHUNK
NPAD = 10112
RPT = NPAD // NSUB


MC = 4
NMEGA = CH // MC


def _make_edge_pass(d):
    mesh = plsc.VectorSubcoreMesh(core_axis_name="c", subcore_axis_name="s")
    MB = 512
    NMB = CH * CHUNK // MB

    @functools.partial(
        pl.kernel,
        out_type=jax.ShapeDtypeStruct((NCORES, NPAD, d), jnp.bfloat16),
        mesh=mesh,
        scratch_types=[
            pltpu.VMEM((CH * CHUNK,), jnp.int32),
            pltpu.VMEM((CH * CHUNK,), jnp.int32),
            pltpu.VMEM((MB, d), jnp.bfloat16),
            pltpu.VMEM((MB, d), jnp.bfloat16),
            pltpu.VMEM_SHARED((NPAD, d), jnp.bfloat16),
            pltpu.SemaphoreType.DMA,
            pltpu.SemaphoreType.DMA,
            pltpu.SemaphoreType.DMA,
            pltpu.SemaphoreType.DMA,
        ],
        compiler_params=pltpu.CompilerParams(use_tc_tiling_on_sc=False),
    )
    def k(g_hbm, src_hbm, dst_hbm, zeros_hbm, out_hbm, src_v, dst_v,
          buf_a, buf_b, acc, gs_a, gs_b, ss_a, ss_b):
        cid = lax.axis_index("c")
        sid = lax.axis_index("s")
        wid = cid * NSUB + sid
        pltpu.sync_copy(zeros_hbm.at[pl.ds(pl.multiple_of(sid * RPT, 8), RPT)],
                        acc.at[pl.ds(pl.multiple_of(sid * RPT, 8), RPT)])
        pltpu.sync_copy(src_hbm.at[pl.ds(wid * CH * CHUNK, CH * CHUNK)],
                        src_v)
        pltpu.sync_copy(dst_hbm.at[pl.ds(wid * CH * CHUNK, CH * CHUNK)],
                        dst_v)
        plsc.subcore_barrier()

        def fire_gather(m, buf, sem):
            pltpu.async_copy(g_hbm.at[src_v.at[pl.ds(m * MB, MB)]], buf, sem)

        def wait_gather(m, buf, sem):
            pltpu.make_async_copy(g_hbm.at[src_v.at[pl.ds(m * MB, MB)]], buf,
                                  sem).wait()

        def fire_scatter(m, buf, sem):
            pltpu.async_copy(buf, acc.at[dst_v.at[pl.ds(m * MB, MB)]], sem,
                             add=True)

        def wait_scatter(m, buf, sem):
            pltpu.make_async_copy(buf, acc.at[dst_v.at[pl.ds(m * MB, MB)]],
                                  sem).wait()

        fire_gather(0, buf_a, gs_a)
        fire_gather(1, buf_b, gs_b)

        def body(i, carry):
            m_a = i * 2
            m_b = i * 2 + 1
            wait_gather(m_a, buf_a, gs_a)
            fire_scatter(m_a, buf_a, ss_a)
            wait_gather(m_b, buf_b, gs_b)
            fire_scatter(m_b, buf_b, ss_b)
            wait_scatter(m_a, buf_a, ss_a)

            @pl.when(i < NMB // 2 - 1)
            def _():
                fire_gather(m_a + 2, buf_a, gs_a)

            wait_scatter(m_b, buf_b, ss_b)

            @pl.when(i < NMB // 2 - 1)
            def _():
                fire_gather(m_b + 2, buf_b, gs_b)

            return carry

        lax.fori_loop(0, NMB // 2, body, 0)
        plsc.subcore_barrier()
        pltpu.sync_copy(acc.at[pl.ds(pl.multiple_of(sid * RPT, 8), RPT)],
                        out_hbm.at[cid].at[pl.ds(pl.multiple_of(sid * RPT, 8), RPT)])

    return k


def _make_deg_pass(d):
    mesh = plsc.VectorSubcoreMesh(core_axis_name="c", subcore_axis_name="s")

    @functools.partial(
        pl.kernel,
        out_type=jax.ShapeDtypeStruct((NCORES, NPAD, d), jnp.float32),
        mesh=mesh,
        scratch_types=[
            pltpu.VMEM((CH, CHUNK), jnp.int32),
            pltpu.VMEM((MC, CHUNK, d), jnp.float32),
            pltpu.VMEM_SHARED((NPAD, d), jnp.float32),
            pltpu.SemaphoreType.DMA,
        ],
        compiler_params=pltpu.CompilerParams(use_tc_tiling_on_sc=False),
    )
    def k(ones_hbm, dst_hbm, zeros_hbm, out_hbm, dst_v, ones_v, acc, sem):
        cid = lax.axis_index("c")
        sid = lax.axis_index("s")
        wid = cid * NSUB + sid
        pltpu.sync_copy(zeros_hbm.at[pl.ds(pl.multiple_of(sid * RPT, 8), RPT)],
                        acc.at[pl.ds(pl.multiple_of(sid * RPT, 8), RPT)])
        pltpu.sync_copy(dst_hbm.at[pl.ds(wid * CH, CH)], dst_v)
        pltpu.sync_copy(ones_hbm, ones_v)
        plsc.subcore_barrier()

        def fire(c, carry):
            pltpu.async_copy(ones_v.at[0], acc.at[dst_v.at[c]], sem, add=True)
            return carry

        def drain(c, carry):
            pltpu.make_async_copy(ones_v.at[0], acc.at[dst_v.at[c]],
                                  sem).wait()
            return carry

        lax.fori_loop(0, CH, fire, 0)
        lax.fori_loop(0, CH, drain, 0)
        plsc.subcore_barrier()
        pltpu.sync_copy(acc.at[pl.ds(pl.multiple_of(sid * RPT, 8), RPT)],
                        out_hbm.at[cid].at[pl.ds(pl.multiple_of(sid * RPT, 8), RPT)])

    return k


_edge_pass_32 = _make_edge_pass(D_HID)
_deg_pass = _make_deg_pass(8)


def _mm1_body(x_ref, w_ref, h_ref):
    h_ref[...] = jnp.dot(x_ref[...], w_ref[...],
                         preferred_element_type=jnp.float32)


def _mm1(x, w1):
    bm = 1000
    return pl.pallas_call(
        _mm1_body,
        grid=(N_NODES // bm,),
        in_specs=[
            pl.BlockSpec((bm, D_IN), lambda i: (i, 0)),
            pl.BlockSpec((D_IN, D_HID), lambda i: (0, 0)),
        ],
        out_specs=pl.BlockSpec((bm, D_HID), lambda i: (i, 0)),
        out_shape=jax.ShapeDtypeStruct((N_NODES, D_HID), jnp.float32),
    )(x, w1)


def _scale_body(h_ref, degp_ref, g_ref, gb_ref, dinv_ref):
    dp = degp_ref[...]
    deg = dp[0, :N_NODES, 0:1] + dp[1, :N_NODES, 0:1] + 1.0
    dinv = 1.0 / jnp.sqrt(deg)
    dinv_ref[...] = dinv
    g = h_ref[...] * dinv
    g_ref[...] = g
    gb_ref[...] = g.astype(jnp.bfloat16)


def _scale(h1, degp):
    return pl.pallas_call(
        _scale_body,
        in_specs=[
            pl.BlockSpec((N_NODES, D_HID), lambda: (0, 0)),
            pl.BlockSpec((NCORES, NPAD, 8), lambda: (0, 0, 0)),
        ],
        out_specs=[
            pl.BlockSpec((N_NODES, D_HID), lambda: (0, 0)),
            pl.BlockSpec((N_NODES, D_HID), lambda: (0, 0)),
            pl.BlockSpec((N_NODES, 1), lambda: (0, 0)),
        ],
        out_shape=[
            jax.ShapeDtypeStruct((N_NODES, D_HID), jnp.float32),
            jax.ShapeDtypeStruct((N_NODES, D_HID), jnp.bfloat16),
            jax.ShapeDtypeStruct((N_NODES, 1), jnp.float32),
        ],
    )(h1, degp)


def _mid_body(part_ref, g1_ref, dinv_ref, b1_ref, f_ref, fb_ref):
    p = part_ref[...].astype(jnp.float32)
    dinv = dinv_ref[...]
    s1 = g1_ref[...] + p[0, :N_NODES, :] + p[1, :N_NODES, :]
    e = jnp.tanh(s1 * dinv + b1_ref[...])
    f = e * dinv
    f_ref[...] = f
    fb_ref[...] = f.astype(jnp.bfloat16)


def _mid(part1, g1, dinv, b1):
    return pl.pallas_call(
        _mid_body,
        in_specs=[
            pl.BlockSpec((NCORES, NPAD, D_HID), lambda: (0, 0, 0)),
            pl.BlockSpec((N_NODES, D_HID), lambda: (0, 0)),
            pl.BlockSpec((N_NODES, 1), lambda: (0, 0)),
            pl.BlockSpec((1, D_HID), lambda: (0, 0)),
        ],
        out_specs=[
            pl.BlockSpec((N_NODES, D_HID), lambda: (0, 0)),
            pl.BlockSpec((N_NODES, D_HID), lambda: (0, 0)),
        ],
        out_shape=[
            jax.ShapeDtypeStruct((N_NODES, D_HID), jnp.float32),
            jax.ShapeDtypeStruct((N_NODES, D_HID), jnp.bfloat16),
        ],
    )(part1, g1, dinv, b1)


def _fin_body(part_ref, f_ref, dinv_ref, wcat_ref, bcat_ref, mu_ref, ls_ref):
    p = part_ref[...].astype(jnp.float32)
    s2 = f_ref[...] + p[0, :N_NODES, :] + p[1, :N_NODES, :]
    h2 = jnp.dot(s2, wcat_ref[...], preferred_element_type=jnp.float32)
    o = h2 * dinv_ref[...] + bcat_ref[...]
    mu_ref[...] = o[:, :D_HID]
    ls_ref[...] = o[:, D_HID:]


def _fin(part2, f, dinv, wcat, bcat):
    return pl.pallas_call(
        _fin_body,
        in_specs=[
            pl.BlockSpec((NCORES, NPAD, D_HID), lambda: (0, 0, 0)),
            pl.BlockSpec((N_NODES, D_HID), lambda: (0, 0)),
            pl.BlockSpec((N_NODES, 1), lambda: (0, 0)),
            pl.BlockSpec((D_HID, 2 * D_HID), lambda: (0, 0)),
            pl.BlockSpec((1, 2 * D_HID), lambda: (0, 0)),
        ],
        out_specs=[
            pl.BlockSpec((N_NODES, D_HID), lambda: (0, 0)),
            pl.BlockSpec((N_NODES, D_HID), lambda: (0, 0)),
        ],
        out_shape=[
            jax.ShapeDtypeStruct((N_NODES, D_HID), jnp.float32),
            jax.ShapeDtypeStruct((N_NODES, D_HID), jnp.float32),
        ],
    )(part2, f, dinv, wcat, bcat)


def kernel(x, edge_index, W1, b1, Wmu, bmu, Wls, bls):
    ei = edge_index.astype(jnp.int32)
    npad = EPAD - N_EDGES
    pad_dst = N_NODES + jnp.arange(npad, dtype=jnp.int32) % (NPAD - N_NODES)
    src1 = jnp.concatenate([ei[0], jnp.zeros((npad,), jnp.int32)])
    dst1 = jnp.concatenate([ei[1], pad_dst])
    dst2d = dst1.reshape(NW * CH, CHUNK)

    z8 = jnp.zeros((NPAD, 8), jnp.float32)
    z32 = jnp.zeros((NPAD, D_HID), jnp.bfloat16)
    ones8 = jnp.ones((MC, CHUNK, 8), jnp.float32)
    wcat = jnp.concatenate([Wmu, Wls], axis=1)
    bcat = jnp.concatenate([bmu, bls]).reshape(1, 2 * D_HID)

    h1 = _mm1(x, W1)
    degp = _deg_pass(ones8, dst2d, z8)
    g1, g1b, dinv = _scale(h1, degp)
    part1 = _edge_pass_32(g1b, src1, dst1, z32)
    f, fb = _mid(part1, g1, dinv, b1.reshape(1, D_HID))
    part2 = _edge_pass_32(fb, src1, dst1, z32)
    mu, ls = _fin(part2, f, dinv, wcat, bcat)
    return mu, ls


# speedup vs baseline: 1.5712x; 1.2671x over previous
import functools

import jax
import jax.numpy as jnp
from jax import lax
from jax.experimental import pallas as pl
from jax.experimental.pallas import tpu as pltpu
from jax.experimental.pallas import tpu_sc as plsc

N_NODES = 10000
N_EDGES = 160000
D_IN = 500
D_HID = 32

NCORES = 2
NSUB = 16
NW = NCORES * NSUB
EW = N_EDGES // NW
MB = 1000
NMB = EW // MB
NPAD = 10112
RPT = NPAD // NSUB


def _make_edge_pass(d):
    mesh = plsc.VectorSubcoreMesh(core_axis_name="c", subcore_axis_name="s")

    @functools.partial(
        pl.kernel,
        out_type=jax.ShapeDtypeStruct((NCORES, NPAD, d), jnp.bfloat16),
        mesh=mesh,
        scratch_types=[
            pltpu.VMEM((EW,), jnp.int32),
            pltpu.VMEM((EW,), jnp.int32),
            pltpu.VMEM((MB, d), jnp.bfloat16),
            pltpu.VMEM((MB, d), jnp.bfloat16),
            pltpu.VMEM_SHARED((NPAD, d), jnp.bfloat16),
            pltpu.SemaphoreType.DMA,
            pltpu.SemaphoreType.DMA,
            pltpu.SemaphoreType.DMA,
            pltpu.SemaphoreType.DMA,
        ],
        compiler_params=pltpu.CompilerParams(use_tc_tiling_on_sc=False),
    )
    def k(g_hbm, src_hbm, dst_hbm, zeros_hbm, out_hbm, src_v, dst_v,
          buf_a, buf_b, acc, gs_a, gs_b, ss_a, ss_b):
        cid = lax.axis_index("c")
        sid = lax.axis_index("s")
        wid = cid * NSUB + sid
        pltpu.sync_copy(zeros_hbm.at[pl.ds(pl.multiple_of(sid * RPT, 8), RPT)],
                        acc.at[pl.ds(pl.multiple_of(sid * RPT, 8), RPT)])
        pltpu.sync_copy(src_hbm.at[pl.ds(wid * EW, EW)], src_v)
        pltpu.sync_copy(dst_hbm.at[pl.ds(wid * EW, EW)], dst_v)
        plsc.subcore_barrier()

        bufs = [buf_a, buf_b]
        gsems = [gs_a, gs_b]
        ssems = [ss_a, ss_b]

        def fire_gather(m):
            pltpu.async_copy(g_hbm.at[src_v.at[pl.ds(m * MB, MB)]],
                             bufs[m % 2], gsems[m % 2])

        def wait_gather(m):
            pltpu.make_async_copy(g_hbm.at[src_v.at[pl.ds(m * MB, MB)]],
                                  bufs[m % 2], gsems[m % 2]).wait()

        def fire_scatter(m):
            pltpu.async_copy(bufs[m % 2], acc.at[dst_v.at[pl.ds(m * MB, MB)]],
                             ssems[m % 2], add=True)

        def wait_scatter(m):
            pltpu.make_async_copy(bufs[m % 2],
                                  acc.at[dst_v.at[pl.ds(m * MB, MB)]],
                                  ssems[m % 2]).wait()

        fire_gather(0)
        fire_gather(1)
        for m in range(NMB):
            wait_gather(m)
            fire_scatter(m)
            if m + 2 < NMB:
                wait_scatter(m)
                fire_gather(m + 2)
        wait_scatter(NMB - 2)
        wait_scatter(NMB - 1)
        plsc.subcore_barrier()
        pltpu.sync_copy(acc.at[pl.ds(pl.multiple_of(sid * RPT, 8), RPT)],
                        out_hbm.at[cid].at[pl.ds(pl.multiple_of(sid * RPT, 8), RPT)])

    return k


def _make_deg_pass(d):
    mesh = plsc.VectorSubcoreMesh(core_axis_name="c", subcore_axis_name="s")

    @functools.partial(
        pl.kernel,
        out_type=jax.ShapeDtypeStruct((NCORES, NPAD, d), jnp.float32),
        mesh=mesh,
        scratch_types=[
            pltpu.VMEM((EW,), jnp.int32),
            pltpu.VMEM((MB, d), jnp.float32),
            pltpu.VMEM_SHARED((NPAD, d), jnp.float32),
            pltpu.SemaphoreType.DMA,
        ],
        compiler_params=pltpu.CompilerParams(use_tc_tiling_on_sc=False),
    )
    def k(ones_hbm, dst_hbm, zeros_hbm, out_hbm, dst_v, ones_v, acc, sem):
        cid = lax.axis_index("c")
        sid = lax.axis_index("s")
        wid = cid * NSUB + sid
        pltpu.sync_copy(zeros_hbm.at[pl.ds(pl.multiple_of(sid * RPT, 8), RPT)],
                        acc.at[pl.ds(pl.multiple_of(sid * RPT, 8), RPT)])
        pltpu.sync_copy(dst_hbm.at[pl.ds(wid * EW, EW)], dst_v)
        pltpu.sync_copy(ones_hbm, ones_v)
        plsc.subcore_barrier()

        for m in range(NMB):
            pltpu.async_copy(ones_v, acc.at[dst_v.at[pl.ds(m * MB, MB)]],
                             sem, add=True)
        for m in range(NMB):
            pltpu.make_async_copy(ones_v,
                                  acc.at[dst_v.at[pl.ds(m * MB, MB)]],
                                  sem).wait()
        plsc.subcore_barrier()
        pltpu.sync_copy(acc.at[pl.ds(pl.multiple_of(sid * RPT, 8), RPT)],
                        out_hbm.at[cid].at[pl.ds(pl.multiple_of(sid * RPT, 8), RPT)])

    return k


_edge_pass_32 = _make_edge_pass(D_HID)
_deg_pass = _make_deg_pass(8)


def _mm1_body(x_ref, w_ref, h_ref):
    h_ref[...] = jnp.dot(x_ref[...], w_ref[...],
                         preferred_element_type=jnp.float32)


def _mm1(x, w1):
    bm = 1000
    return pl.pallas_call(
        _mm1_body,
        grid=(N_NODES // bm,),
        in_specs=[
            pl.BlockSpec((bm, D_IN), lambda i: (i, 0)),
            pl.BlockSpec((D_IN, D_HID), lambda i: (0, 0)),
        ],
        out_specs=pl.BlockSpec((bm, D_HID), lambda i: (i, 0)),
        out_shape=jax.ShapeDtypeStruct((N_NODES, D_HID), jnp.float32),
    )(x, w1)


def _scale_body(h_ref, degp_ref, g_ref, gb_ref, dinv_ref):
    dp = degp_ref[...]
    deg = dp[0, :N_NODES, 0:1] + dp[1, :N_NODES, 0:1] + 1.0
    dinv = 1.0 / jnp.sqrt(deg)
    dinv_ref[...] = dinv
    g = h_ref[...] * dinv
    g_ref[...] = g
    gb_ref[...] = g.astype(jnp.bfloat16)


def _scale(h1, degp):
    return pl.pallas_call(
        _scale_body,
        in_specs=[
            pl.BlockSpec((N_NODES, D_HID), lambda: (0, 0)),
            pl.BlockSpec((NCORES, NPAD, 8), lambda: (0, 0, 0)),
        ],
        out_specs=[
            pl.BlockSpec((N_NODES, D_HID), lambda: (0, 0)),
            pl.BlockSpec((N_NODES, D_HID), lambda: (0, 0)),
            pl.BlockSpec((N_NODES, 1), lambda: (0, 0)),
        ],
        out_shape=[
            jax.ShapeDtypeStruct((N_NODES, D_HID), jnp.float32),
            jax.ShapeDtypeStruct((N_NODES, D_HID), jnp.bfloat16),
            jax.ShapeDtypeStruct((N_NODES, 1), jnp.float32),
        ],
    )(h1, degp)


def _mid_body(part_ref, g1_ref, dinv_ref, b1_ref, f_ref, fb_ref):
    p = part_ref[...].astype(jnp.float32)
    dinv = dinv_ref[...]
    s1 = g1_ref[...] + p[0, :N_NODES, :] + p[1, :N_NODES, :]
    e = jnp.tanh(s1 * dinv + b1_ref[...])
    f = e * dinv
    f_ref[...] = f
    fb_ref[...] = f.astype(jnp.bfloat16)


def _mid(part1, g1, dinv, b1):
    return pl.pallas_call(
        _mid_body,
        in_specs=[
            pl.BlockSpec((NCORES, NPAD, D_HID), lambda: (0, 0, 0)),
            pl.BlockSpec((N_NODES, D_HID), lambda: (0, 0)),
            pl.BlockSpec((N_NODES, 1), lambda: (0, 0)),
            pl.BlockSpec((1, D_HID), lambda: (0, 0)),
        ],
        out_specs=[
            pl.BlockSpec((N_NODES, D_HID), lambda: (0, 0)),
            pl.BlockSpec((N_NODES, D_HID), lambda: (0, 0)),
        ],
        out_shape=[
            jax.ShapeDtypeStruct((N_NODES, D_HID), jnp.float32),
            jax.ShapeDtypeStruct((N_NODES, D_HID), jnp.bfloat16),
        ],
    )(part1, g1, dinv, b1)


def _fin_body(part_ref, f_ref, dinv_ref, wcat_ref, bcat_ref, mu_ref, ls_ref):
    p = part_ref[...].astype(jnp.float32)
    s2 = f_ref[...] + p[0, :N_NODES, :] + p[1, :N_NODES, :]
    h2 = jnp.dot(s2, wcat_ref[...], preferred_element_type=jnp.float32)
    o = h2 * dinv_ref[...] + bcat_ref[...]
    mu_ref[...] = o[:, :D_HID]
    ls_ref[...] = o[:, D_HID:]


def _fin(part2, f, dinv, wcat, bcat):
    return pl.pallas_call(
        _fin_body,
        in_specs=[
            pl.BlockSpec((NCORES, NPAD, D_HID), lambda: (0, 0, 0)),
            pl.BlockSpec((N_NODES, D_HID), lambda: (0, 0)),
            pl.BlockSpec((N_NODES, 1), lambda: (0, 0)),
            pl.BlockSpec((D_HID, 2 * D_HID), lambda: (0, 0)),
            pl.BlockSpec((1, 2 * D_HID), lambda: (0, 0)),
        ],
        out_specs=[
            pl.BlockSpec((N_NODES, D_HID), lambda: (0, 0)),
            pl.BlockSpec((N_NODES, D_HID), lambda: (0, 0)),
        ],
        out_shape=[
            jax.ShapeDtypeStruct((N_NODES, D_HID), jnp.float32),
            jax.ShapeDtypeStruct((N_NODES, D_HID), jnp.float32),
        ],
    )(part2, f, dinv, wcat, bcat)


def kernel(x, edge_index, W1, b1, Wmu, bmu, Wls, bls):
    ei = edge_index.astype(jnp.int32)
    src1 = ei[0]
    dst1 = ei[1]

    z8 = jnp.zeros((NPAD, 8), jnp.float32)
    z32 = jnp.zeros((NPAD, D_HID), jnp.bfloat16)
    ones8 = jnp.ones((MB, 8), jnp.float32)
    wcat = jnp.concatenate([Wmu, Wls], axis=1)
    bcat = jnp.concatenate([bmu, bls]).reshape(1, 2 * D_HID)

    h1 = _mm1(x, W1)
    degp = _deg_pass(ones8, dst1, z8)
    g1, g1b, dinv = _scale(h1, degp)
    part1 = _edge_pass_32(g1b, src1, dst1, z32)
    f, fb = _mid(part1, g1, dinv, b1.reshape(1, D_HID))
    part2 = _edge_pass_32(fb, src1, dst1, z32)
    mu, ls = _fin(part2, f, dinv, wcat, bcat)
    return mu, ls


# speedup vs baseline: 1.5990x; 1.0176x over previous
import functools

import jax
import jax.numpy as jnp
from jax import lax
from jax.experimental import pallas as pl
from jax.experimental.pallas import tpu as pltpu
from jax.experimental.pallas import tpu_sc as plsc

N_NODES = 10000
N_EDGES = 160000
D_IN = 500
D_HID = 32

NCORES = 2
NSUB = 16
NW = NCORES * NSUB
EW = N_EDGES // NW
MB = 1000
NMB = EW // MB
NPAD = 10112
RPT = NPAD // NSUB


def _make_edge_pass(d):
    mesh = plsc.VectorSubcoreMesh(core_axis_name="c", subcore_axis_name="s")

    @functools.partial(
        pl.kernel,
        out_type=jax.ShapeDtypeStruct((NCORES, NPAD, d), jnp.bfloat16),
        mesh=mesh,
        scratch_types=[
            pltpu.VMEM((EW,), jnp.int32),
            pltpu.VMEM((EW,), jnp.int32),
            pltpu.VMEM((NMB, MB, d), jnp.bfloat16),
            pltpu.VMEM_SHARED((NPAD, d), jnp.bfloat16),
            pltpu.SemaphoreType.DMA,
            pltpu.SemaphoreType.DMA,
        ],
        compiler_params=pltpu.CompilerParams(use_tc_tiling_on_sc=False),
    )
    def k(g_hbm, src_hbm, dst_hbm, zeros_hbm, out_hbm, src_v, dst_v,
          bufs, acc, gsem, ssem):
        cid = lax.axis_index("c")
        sid = lax.axis_index("s")
        wid = cid * NSUB + sid
        pltpu.sync_copy(zeros_hbm.at[pl.ds(pl.multiple_of(sid * RPT, 8), RPT)],
                        acc.at[pl.ds(pl.multiple_of(sid * RPT, 8), RPT)])
        pltpu.sync_copy(src_hbm.at[pl.ds(wid * EW, EW)], src_v)
        pltpu.sync_copy(dst_hbm.at[pl.ds(wid * EW, EW)], dst_v)
        plsc.subcore_barrier()

        for m in range(NMB):
            pltpu.async_copy(g_hbm.at[src_v.at[pl.ds(m * MB, MB)]],
                             bufs.at[m], gsem)
        for m in range(NMB):
            pltpu.make_async_copy(g_hbm.at[src_v.at[pl.ds(m * MB, MB)]],
                                  bufs.at[m], gsem).wait()
            pltpu.async_copy(bufs.at[m], acc.at[dst_v.at[pl.ds(m * MB, MB)]],
                             ssem, add=True)
        for m in range(NMB):
            pltpu.make_async_copy(bufs.at[m],
                                  acc.at[dst_v.at[pl.ds(m * MB, MB)]],
                                  ssem).wait()
        plsc.subcore_barrier()
        pltpu.sync_copy(acc.at[pl.ds(pl.multiple_of(sid * RPT, 8), RPT)],
                        out_hbm.at[cid].at[pl.ds(pl.multiple_of(sid * RPT, 8), RPT)])

    return k


def _make_deg_pass(d):
    mesh = plsc.VectorSubcoreMesh(core_axis_name="c", subcore_axis_name="s")

    @functools.partial(
        pl.kernel,
        out_type=jax.ShapeDtypeStruct((NCORES, NPAD, d), jnp.float32),
        mesh=mesh,
        scratch_types=[
            pltpu.VMEM((EW,), jnp.int32),
            pltpu.VMEM((MB, d), jnp.float32),
            pltpu.VMEM_SHARED((NPAD, d), jnp.float32),
            pltpu.SemaphoreType.DMA,
        ],
        compiler_params=pltpu.CompilerParams(use_tc_tiling_on_sc=False),
    )
    def k(ones_hbm, dst_hbm, zeros_hbm, out_hbm, dst_v, ones_v, acc, sem):
        cid = lax.axis_index("c")
        sid = lax.axis_index("s")
        wid = cid * NSUB + sid
        pltpu.sync_copy(zeros_hbm.at[pl.ds(pl.multiple_of(sid * RPT, 8), RPT)],
                        acc.at[pl.ds(pl.multiple_of(sid * RPT, 8), RPT)])
        pltpu.sync_copy(dst_hbm.at[pl.ds(wid * EW, EW)], dst_v)
        pltpu.sync_copy(ones_hbm, ones_v)
        plsc.subcore_barrier()

        for m in range(NMB):
            pltpu.async_copy(ones_v, acc.at[dst_v.at[pl.ds(m * MB, MB)]],
                             sem, add=True)
        for m in range(NMB):
            pltpu.make_async_copy(ones_v,
                                  acc.at[dst_v.at[pl.ds(m * MB, MB)]],
                                  sem).wait()
        plsc.subcore_barrier()
        pltpu.sync_copy(acc.at[pl.ds(pl.multiple_of(sid * RPT, 8), RPT)],
                        out_hbm.at[cid].at[pl.ds(pl.multiple_of(sid * RPT, 8), RPT)])

    return k


_edge_pass_32 = _make_edge_pass(D_HID)
_deg_pass = _make_deg_pass(8)


def _mm1_body(x_ref, w_ref, h_ref):
    h_ref[...] = jnp.dot(x_ref[...], w_ref[...],
                         preferred_element_type=jnp.float32)


def _mm1(x, w1):
    bm = 1000
    return pl.pallas_call(
        _mm1_body,
        grid=(N_NODES // bm,),
        in_specs=[
            pl.BlockSpec((bm, D_IN), lambda i: (i, 0)),
            pl.BlockSpec((D_IN, D_HID), lambda i: (0, 0)),
        ],
        out_specs=pl.BlockSpec((bm, D_HID), lambda i: (i, 0)),
        out_shape=jax.ShapeDtypeStruct((N_NODES, D_HID), jnp.float32),
    )(x, w1)


def _scale_body(h_ref, degp_ref, g_ref, gb_ref, dinv_ref):
    dp = degp_ref[...]
    deg = dp[0, :N_NODES, 0:1] + dp[1, :N_NODES, 0:1] + 1.0
    dinv = 1.0 / jnp.sqrt(deg)
    dinv_ref[...] = dinv
    g = h_ref[...] * dinv
    g_ref[...] = g
    gb_ref[...] = g.astype(jnp.bfloat16)


def _scale(h1, degp):
    return pl.pallas_call(
        _scale_body,
        in_specs=[
            pl.BlockSpec((N_NODES, D_HID), lambda: (0, 0)),
            pl.BlockSpec((NCORES, NPAD, 8), lambda: (0, 0, 0)),
        ],
        out_specs=[
            pl.BlockSpec((N_NODES, D_HID), lambda: (0, 0)),
            pl.BlockSpec((N_NODES, D_HID), lambda: (0, 0)),
            pl.BlockSpec((N_NODES, 1), lambda: (0, 0)),
        ],
        out_shape=[
            jax.ShapeDtypeStruct((N_NODES, D_HID), jnp.float32),
            jax.ShapeDtypeStruct((N_NODES, D_HID), jnp.bfloat16),
            jax.ShapeDtypeStruct((N_NODES, 1), jnp.float32),
        ],
    )(h1, degp)


def _mid_body(part_ref, g1_ref, dinv_ref, b1_ref, f_ref, fb_ref):
    p = part_ref[...].astype(jnp.float32)
    dinv = dinv_ref[...]
    s1 = g1_ref[...] + p[0, :N_NODES, :] + p[1, :N_NODES, :]
    e = jnp.tanh(s1 * dinv + b1_ref[...])
    f = e * dinv
    f_ref[...] = f
    fb_ref[...] = f.astype(jnp.bfloat16)


def _mid(part1, g1, dinv, b1):
    return pl.pallas_call(
        _mid_body,
        in_specs=[
            pl.BlockSpec((NCORES, NPAD, D_HID), lambda: (0, 0, 0)),
            pl.BlockSpec((N_NODES, D_HID), lambda: (0, 0)),
            pl.BlockSpec((N_NODES, 1), lambda: (0, 0)),
            pl.BlockSpec((1, D_HID), lambda: (0, 0)),
        ],
        out_specs=[
            pl.BlockSpec((N_NODES, D_HID), lambda: (0, 0)),
            pl.BlockSpec((N_NODES, D_HID), lambda: (0, 0)),
        ],
        out_shape=[
            jax.ShapeDtypeStruct((N_NODES, D_HID), jnp.float32),
            jax.ShapeDtypeStruct((N_NODES, D_HID), jnp.bfloat16),
        ],
    )(part1, g1, dinv, b1)


def _fin_body(part_ref, f_ref, dinv_ref, wcat_ref, bcat_ref, mu_ref, ls_ref):
    p = part_ref[...].astype(jnp.float32)
    s2 = f_ref[...] + p[0, :N_NODES, :] + p[1, :N_NODES, :]
    h2 = jnp.dot(s2, wcat_ref[...], preferred_element_type=jnp.float32)
    o = h2 * dinv_ref[...] + bcat_ref[...]
    mu_ref[...] = o[:, :D_HID]
    ls_ref[...] = o[:, D_HID:]


def _fin(part2, f, dinv, wcat, bcat):
    return pl.pallas_call(
        _fin_body,
        in_specs=[
            pl.BlockSpec((NCORES, NPAD, D_HID), lambda: (0, 0, 0)),
            pl.BlockSpec((N_NODES, D_HID), lambda: (0, 0)),
            pl.BlockSpec((N_NODES, 1), lambda: (0, 0)),
            pl.BlockSpec((D_HID, 2 * D_HID), lambda: (0, 0)),
            pl.BlockSpec((1, 2 * D_HID), lambda: (0, 0)),
        ],
        out_specs=[
            pl.BlockSpec((N_NODES, D_HID), lambda: (0, 0)),
            pl.BlockSpec((N_NODES, D_HID), lambda: (0, 0)),
        ],
        out_shape=[
            jax.ShapeDtypeStruct((N_NODES, D_HID), jnp.float32),
            jax.ShapeDtypeStruct((N_NODES, D_HID), jnp.float32),
        ],
    )(part2, f, dinv, wcat, bcat)


def kernel(x, edge_index, W1, b1, Wmu, bmu, Wls, bls):
    ei = edge_index.astype(jnp.int32)
    src1 = ei[0]
    dst1 = ei[1]

    z8 = jnp.zeros((NPAD, 8), jnp.float32)
    z32 = jnp.zeros((NPAD, D_HID), jnp.bfloat16)
    ones8 = jnp.ones((MB, 8), jnp.float32)
    wcat = jnp.concatenate([Wmu, Wls], axis=1)
    bcat = jnp.concatenate([bmu, bls]).reshape(1, 2 * D_HID)

    h1 = _mm1(x, W1)
    degp = _deg_pass(ones8, dst1, z8)
    g1, g1b, dinv = _scale(h1, degp)
    part1 = _edge_pass_32(g1b, src1, dst1, z32)
    f, fb = _mid(part1, g1, dinv, b1.reshape(1, D_HID))
    part2 = _edge_pass_32(fb, src1, dst1, z32)
    mu, ls = _fin(part2, f, dinv, wcat, bcat)
    return mu, ls
